# Initial kernel scaffold; baseline (speedup 1.0000x reference)
import jax
import jax.numpy as jnp
from jax.experimental import pallas as pl


def kernel(input_ids, table):
    raise NotImplementedError("write your pallas kernel here")


# speedup vs baseline: 1.7359x; 1.7359x over previous
import functools

import jax
import jax.numpy as jnp
from jax import lax
from jax.experimental import pallas as pl
from jax.experimental.pallas import tpu as pltpu
from jax.experimental.pallas import tpu_sc as plsc

NUM_BUCKETS = 1000000
D = 32
ROWS, COLS = 16384, 26
B = ROWS * COLS
NC, NS, L = 2, 16, 16
NW = NC * NS
BPW = B // NW
IW = 128
NROW = BPW // IW
GCH = 8
GROUP = GCH * IW
NG = NROW // GCH
NBUF = 3


def _body(ids_hbm, table_hbm, out_hbm, idx_v, rb0, rb1, rb2,
          gs0, gs1, gs2, ps0, ps1, ps2):
    rbufs = (rb0, rb1, rb2)
    gsems = (gs0, gs1, gs2)
    psems = (ps0, ps1, ps2)
    wid = lax.axis_index("s") * NC + lax.axis_index("c")
    base = wid * BPW

    pltpu.sync_copy(ids_hbm.at[wid], idx_v)

    def hash_row(j, carry):
        c10 = jnp.int32(10)
        c1023 = jnp.int32(1023)
        cm1 = jnp.int32(219264)
        cm2 = jnp.int32(435761)
        cmod = jnp.int32(NUM_BUCKETS)
        for u in range(IW // L):
            v = idx_v[j, pl.ds(u * L, L)]
            a = lax.shift_right_logical(v, c10)
            b = lax.bitwise_and(v, c1023)
            s = a * cm1 + b * cm2
            idx_v[j, pl.ds(u * L, L)] = lax.rem(s, cmod)
        return carry

    lax.fori_loop(0, NROW, hash_row, 0)

    gcopies = [None] * NG
    pcopies = [None] * NG

    def fire_gathers(g):
        buf = rbufs[g % NBUF]
        sem = gsems[g % NBUF]
        cs = []
        for r in range(GCH):
            j = g * GCH + r
            cs.append(pltpu.async_copy(
                table_hbm.at[idx_v.at[jnp.int32(j)]],
                buf.at[pl.ds(r * IW, IW)], sem))
        return cs

    def fire_putout(g):
        return pltpu.async_copy(
            rbufs[g % NBUF],
            out_hbm.at[pl.ds(base + g * GROUP, GROUP)],
            psems[g % NBUF])

    for g in range(NG):
        if g >= NBUF:
            pcopies[g - NBUF].wait()
        gcopies[g] = fire_gathers(g)
        if g >= 1:
            for c in gcopies[g - 1]:
                c.wait()
            pcopies[g - 1] = fire_putout(g - 1)
    for c in gcopies[NG - 1]:
        c.wait()
    pcopies[NG - 1] = fire_putout(NG - 1)
    for g in range(NG - NBUF, NG):
        pcopies[g].wait()


@functools.partial(jax.jit, static_argnames=())
def _sc_gather(ids3, table):
    mesh = plsc.VectorSubcoreMesh(core_axis_name="c", subcore_axis_name="s")
    kfn = pl.kernel(
        _body,
        out_type=jax.ShapeDtypeStruct((B, D), jnp.float32),
        mesh=mesh,
        compiler_params=pltpu.CompilerParams(use_tc_tiling_on_sc=False),
        scratch_types=[
            pltpu.VMEM((NROW, IW), jnp.int32),
            pltpu.VMEM((GROUP, D), jnp.float32),
            pltpu.VMEM((GROUP, D), jnp.float32),
            pltpu.VMEM((GROUP, D), jnp.float32),
            pltpu.SemaphoreType.DMA,
            pltpu.SemaphoreType.DMA,
            pltpu.SemaphoreType.DMA,
            pltpu.SemaphoreType.DMA,
            pltpu.SemaphoreType.DMA,
            pltpu.SemaphoreType.DMA,
        ],
    )
    return kfn(ids3, table)


def kernel(input_ids, table):
    ids = input_ids.reshape(-1).astype(jnp.int32).reshape(NW, NROW, IW)
    out = _sc_gather(ids, table.astype(jnp.float32))
    return out.reshape(ROWS, COLS, D)


# speedup vs baseline: 2.0890x; 1.2034x over previous
import functools

import jax
import jax.numpy as jnp
from jax import lax
from jax.experimental import pallas as pl
from jax.experimental.pallas import tpu as pltpu
from jax.experimental.pallas import tpu_sc as plsc

NUM_BUCKETS = 1000000
D = 32
ROWS, COLS = 16384, 26
B = ROWS * COLS
NC, NS, L = 2, 16, 16
NW = NC * NS
BPW = B // NW
IW = 128
NBLK = BPW // IW
TCL = 4096
Q = TCL // 4
TC_GRID = (NUM_BUCKETS + TCL - 1) // TCL
R_ROWS = TC_GRID * Q


def _relayout_body(x_ref, o_ref):
    for q in range(4):
        o_ref[:, q * 32:(q + 1) * 32] = jnp.transpose(
            x_ref[:, q * Q:(q + 1) * Q], (1, 0))


def _tc_relayout(table_t):
    return pl.pallas_call(
        _relayout_body,
        grid=(TC_GRID,),
        in_specs=[pl.BlockSpec((32, TCL), lambda k: (jnp.int32(0), k))],
        out_specs=pl.BlockSpec((Q, 128), lambda k: (k, jnp.int32(0))),
        out_shape=jax.ShapeDtypeStruct((R_ROWS, 128), jnp.float32),
    )(table_t)


def _sc_body(ids_hbm, trelay_hbm, out_hbm, idx_v, sub_v, gb0, gb1, ob0, ob1,
             cst_v, gs0, gs1, os0, os1):
    gbufs = (gb0, gb1)
    obufs = (ob0, ob1)
    gsems = (gs0, gs1)
    osems = (os0, os1)
    wid = lax.axis_index("s") * NC + lax.axis_index("c")
    qbase = wid * BPW

    pltpu.sync_copy(ids_hbm.at[pl.ds(wid * NBLK, NBLK)], idx_v)

    iota = lax.iota(jnp.int32, L)
    cst_v[0, :] = iota
    cst_v[1, :] = iota * jnp.int32(128)

    def hash_row(j, carry):
        c10 = jnp.int32(10)
        c1023 = jnp.int32(1023)
        cm1 = jnp.int32(219264)
        cm2 = jnp.int32(435761)
        cmod = jnp.int32(NUM_BUCKETS)
        for g in range(IW // L):
            v = idx_v[j, pl.ds(g * L, L)]
            a = lax.shift_right_logical(v, c10)
            b = lax.bitwise_and(v, c1023)
            h = lax.rem(a * cm1 + b * cm2, cmod)
            row = lax.bitwise_or(
                lax.shift_left(lax.shift_right_logical(h, jnp.int32(12)),
                               jnp.int32(10)),
                lax.bitwise_and(h, c1023))
            idx_v[j, pl.ds(g * L, L)] = row
            sub_v[j, pl.ds(g * L, L)] = lax.shift_left(
                lax.bitwise_and(lax.shift_right_logical(h, c10), jnp.int32(3)),
                jnp.int32(5))
        return carry

    lax.fori_loop(jnp.int32(0), jnp.int32(NBLK), hash_row, 0)

    def fire_gather(j, par):
        return pltpu.async_copy(
            trelay_hbm.at[idx_v.at[j]], gbufs[par], gsems[par])

    def wait_gather(par):
        pltpu.make_async_copy(
            trelay_hbm.at[idx_v.at[jnp.int32(0)]], gbufs[par],
            gsems[par]).wait()

    def out_slice(j):
        qb = qbase + j * IW
        c = lax.shift_right_logical(qb, jnp.int32(14))
        r = pl.multiple_of(lax.bitwise_and(qb, jnp.int32(16383)), IW)
        return out_hbm.at[c, :, pl.ds(r, IW)]

    def fire_out(j, par):
        return pltpu.async_copy(obufs[par], out_slice(j), osems[par])

    def wait_out(par):
        pltpu.make_async_copy(obufs[par], out_slice(jnp.int32(0)),
                              osems[par]).wait()

    def extract(j, par):
        gbuf = gbufs[par]
        obuf = obufs[par]
        iota_v = cst_v[0, :]
        for g in range(IW // L):
            rowv = jnp.int32(g * L) + iota_v
            subv = sub_v[j, pl.ds(g * L, L)]
            for d in range(D):
                vals = plsc.load_gather(gbuf, [rowv, subv + jnp.int32(d)])
                obuf[d, pl.ds(g * L, L)] = vals

    fire_gather(jnp.int32(0), 0)
    fire_gather(jnp.int32(1), 1)

    def step(jj, carry):
        for par in range(2):
            j = jj * jnp.int32(2) + jnp.int32(par)
            wait_gather(par)

            @pl.when(jj >= 1)
            def _():
                wait_out(par)

            extract(j, par)
            fire_out(j, par)

            @pl.when(jj <= NBLK // 2 - 2)
            def _():
                fire_gather(j + jnp.int32(2), par)
        return carry

    lax.fori_loop(jnp.int32(0), jnp.int32(NBLK // 2), step, 0)
    wait_out(0)
    wait_out(1)


@jax.jit
def _sc_gather(ids2d, trelay):
    mesh = plsc.VectorSubcoreMesh(core_axis_name="c", subcore_axis_name="s")
    kfn = pl.kernel(
        _sc_body,
        out_type=jax.ShapeDtypeStruct((COLS, D, ROWS), jnp.float32),
        mesh=mesh,
        compiler_params=pltpu.CompilerParams(
            use_tc_tiling_on_sc=True, needs_layout_passes=False),
        scratch_types=[
            pltpu.VMEM((NBLK, IW), jnp.int32),
            pltpu.VMEM((NBLK, IW), jnp.int32),
            pltpu.VMEM((IW, 128), jnp.float32),
            pltpu.VMEM((IW, 128), jnp.float32),
            pltpu.VMEM((D, IW), jnp.float32),
            pltpu.VMEM((D, IW), jnp.float32),
            pltpu.VMEM((2, L), jnp.int32),
            pltpu.SemaphoreType.DMA,
            pltpu.SemaphoreType.DMA,
            pltpu.SemaphoreType.DMA,
            pltpu.SemaphoreType.DMA,
        ],
    )
    return kfn(ids2d, trelay)


def kernel(input_ids, table):
    ids2d = jnp.transpose(input_ids).astype(jnp.int32).reshape(B // IW, IW)
    table_t = jnp.transpose(table)
    trelay = _tc_relayout(table_t)
    out_t = _sc_gather(ids2d, trelay)
    return jnp.transpose(out_t, (2, 0, 1))
